# baseline (speedup 1.0000x reference)
import functools

import jax
import jax.numpy as jnp
from jax import lax
from jax.experimental import pallas as pl
from jax.experimental.pallas import tpu as pltpu
from jax.experimental.pallas import tpu_sc as plsc

_B = 16384
_E = 128
_NC = 2
_NS = 16
_NW = _NC * _NS
_PW = _B // _NW
_C = 128
_NCHUNK = _PW // _C
_NBUF = 3


def _sc_body(xt_hbm, xc_hbm, wt_hbm, wc_hbm, out_hbm,
             idxt_v, idxc_v, rt0, rc0, rt1, rc1, rt2, rc2, out_v,
             sem0, sem1, sem2):
    wid = lax.axis_index("s") * _NC + lax.axis_index("c")
    base = wid * _PW

    rows = [(rt0, rc0), (rt1, rc1), (rt2, rc2)]
    sems = [sem0, sem1, sem2]

    def start(c):
        rt, rc = rows[c % _NBUF]
        sem = sems[c % _NBUF]
        cp_t = pltpu.async_copy(wt_hbm.at[idxt_v.at[pl.ds(c * _C, _C)]], rt, sem)
        cp_c = pltpu.async_copy(wc_hbm.at[idxc_v.at[pl.ds(c * _C, _C)]], rc, sem)
        return cp_t, cp_c

    pltpu.sync_copy(xt_hbm.at[pl.ds(base, _C)], idxt_v.at[pl.ds(0, _C)])
    pltpu.sync_copy(xc_hbm.at[pl.ds(base, _C)], idxc_v.at[pl.ds(0, _C)])
    inflight = [start(0)]
    pltpu.sync_copy(xt_hbm.at[pl.ds(base + _C, _PW - _C)],
                    idxt_v.at[pl.ds(_C, _PW - _C)])
    pltpu.sync_copy(xc_hbm.at[pl.ds(base + _C, _PW - _C)],
                    idxc_v.at[pl.ds(_C, _PW - _C)])
    inflight.append(start(1))

    lane = lax.iota(jnp.int32, 16)
    perms = [lane ^ jnp.int32(1 << p) for p in (3, 2, 1, 0)]

    for c in range(_NCHUNK):
        for cp in inflight[0]:
            cp.wait()
        inflight.pop(0)
        if c + 2 < _NCHUNK:
            inflight.append(start(c + 2))
        rt, rc = rows[c % _NBUF]

        def grp_body(g, carry, rt=rt, rc=rc, c=c):
            def tok_body(i, res, rt=rt, rc=rc):
                t = g * 16 + i
                s = rt[t, pl.ds(0, 16)] * rc[t, pl.ds(0, 16)]
                for k in range(1, _E // 16):
                    s = s + (rt[t, pl.ds(k * 16, 16)]
                             * rc[t, pl.ds(k * 16, 16)])
                for pv in perms:
                    s = s + s.at[pv].get(mode="promise_in_bounds")
                return jnp.where(lane == i, s, res)

            res = lax.fori_loop(0, 16, tok_body,
                                jnp.zeros((16,), jnp.float32), unroll=8)
            out_v[pl.ds(c * _C + g * 16, 16)] = res
            return carry

        lax.fori_loop(0, _C // 16, grp_body, 0)

    pltpu.sync_copy(out_v, out_hbm.at[pl.ds(base, _PW)])


@jax.jit
def _skipgram_sc(xt, xc, wt, wc):
    f = functools.partial(
        pl.kernel,
        out_type=jax.ShapeDtypeStruct((_B,), jnp.float32),
        mesh=plsc.VectorSubcoreMesh(core_axis_name="c", subcore_axis_name="s"),
        scratch_types=[
            pltpu.VMEM((_PW,), jnp.int32),
            pltpu.VMEM((_PW,), jnp.int32),
            pltpu.VMEM((_C, _E), jnp.float32),
            pltpu.VMEM((_C, _E), jnp.float32),
            pltpu.VMEM((_C, _E), jnp.float32),
            pltpu.VMEM((_C, _E), jnp.float32),
            pltpu.VMEM((_C, _E), jnp.float32),
            pltpu.VMEM((_C, _E), jnp.float32),
            pltpu.VMEM((_PW,), jnp.float32),
            pltpu.SemaphoreType.DMA,
            pltpu.SemaphoreType.DMA,
            pltpu.SemaphoreType.DMA,
        ],
    )(_sc_body)
    return f(xt, xc, wt, wc)


def kernel(X_target, X_context, W_target, W_context):
    xt = X_target.astype(jnp.int32)
    xc = X_context.astype(jnp.int32)
    return _skipgram_sc(xt, xc, W_target, W_context)


# speedup vs baseline: 1.0103x; 1.0103x over previous
import functools

import jax
import jax.numpy as jnp
from jax import lax
from jax.experimental import pallas as pl
from jax.experimental.pallas import tpu as pltpu
from jax.experimental.pallas import tpu_sc as plsc

_B = 16384
_E = 128
_NC = 2
_NS = 16
_NW = _NC * _NS
_PW = _B // _NW
_C = 128
_NCHUNK = _PW // _C
_NBUF = 3


def _sc_body(xt_hbm, xc_hbm, wt_hbm, wc_hbm, out_hbm,
             idxt_v, idxc_v, rt0, rc0, rt1, rc1, rt2, rc2, out_v,
             sem0, sem1, sem2):
    wid = lax.axis_index("s") * _NC + lax.axis_index("c")
    base = wid * _PW

    rows = [(rt0, rc0), (rt1, rc1), (rt2, rc2)]
    sems = [sem0, sem1, sem2]

    def start(c):
        rt, rc = rows[c % _NBUF]
        sem = sems[c % _NBUF]
        cp_t = pltpu.async_copy(wt_hbm.at[idxt_v.at[pl.ds(c * _C, _C)]], rt, sem)
        cp_c = pltpu.async_copy(wc_hbm.at[idxc_v.at[pl.ds(c * _C, _C)]], rc, sem)
        return cp_t, cp_c

    pltpu.sync_copy(xt_hbm.at[pl.ds(base, _C)], idxt_v.at[pl.ds(0, _C)])
    pltpu.sync_copy(xc_hbm.at[pl.ds(base, _C)], idxc_v.at[pl.ds(0, _C)])
    inflight = [start(0)]
    pltpu.sync_copy(xt_hbm.at[pl.ds(base + _C, _PW - _C)],
                    idxt_v.at[pl.ds(_C, _PW - _C)])
    pltpu.sync_copy(xc_hbm.at[pl.ds(base + _C, _PW - _C)],
                    idxc_v.at[pl.ds(_C, _PW - _C)])
    inflight.append(start(1))
    inflight.append(start(2))

    lane = lax.iota(jnp.int32, 16)
    perms = [lane ^ jnp.int32(1 << p) for p in (3, 2, 1, 0)]

    for c in range(_NCHUNK):
        for cp in inflight[0]:
            cp.wait()
        inflight.pop(0)
        rt, rc = rows[c % _NBUF]

        def grp_body(g, carry, rt=rt, rc=rc, c=c):
            def tok_body(i, res, rt=rt, rc=rc):
                t = g * 16 + i
                s = rt[t, pl.ds(0, 16)] * rc[t, pl.ds(0, 16)]
                for k in range(1, _E // 16):
                    s = s + (rt[t, pl.ds(k * 16, 16)]
                             * rc[t, pl.ds(k * 16, 16)])
                for pv in perms:
                    s = s + s.at[pv].get(mode="promise_in_bounds")
                return jnp.where(lane == i, s, res)

            res = lax.fori_loop(0, 16, tok_body,
                                jnp.zeros((16,), jnp.float32), unroll=4)
            out_v[pl.ds(c * _C + g * 16, 16)] = res
            return carry

        lax.fori_loop(0, _C // 16, grp_body, 0)
        if c + 3 < _NCHUNK:
            inflight.append(start(c + 3))

    pltpu.sync_copy(out_v, out_hbm.at[pl.ds(base, _PW)])


@jax.jit
def _skipgram_sc(xt, xc, wt, wc):
    f = functools.partial(
        pl.kernel,
        out_type=jax.ShapeDtypeStruct((_B,), jnp.float32),
        mesh=plsc.VectorSubcoreMesh(core_axis_name="c", subcore_axis_name="s"),
        scratch_types=[
            pltpu.VMEM((_PW,), jnp.int32),
            pltpu.VMEM((_PW,), jnp.int32),
            pltpu.VMEM((_C, _E), jnp.float32),
            pltpu.VMEM((_C, _E), jnp.float32),
            pltpu.VMEM((_C, _E), jnp.float32),
            pltpu.VMEM((_C, _E), jnp.float32),
            pltpu.VMEM((_C, _E), jnp.float32),
            pltpu.VMEM((_C, _E), jnp.float32),
            pltpu.VMEM((_PW,), jnp.float32),
            pltpu.SemaphoreType.DMA,
            pltpu.SemaphoreType.DMA,
            pltpu.SemaphoreType.DMA,
        ],
    )(_sc_body)
    return f(xt, xc, wt, wc)


def kernel(X_target, X_context, W_target, W_context):
    xt = X_target.astype(jnp.int32)
    xc = X_context.astype(jnp.int32)
    return _skipgram_sc(xt, xc, W_target, W_context)


# speedup vs baseline: 1.0561x; 1.0453x over previous
import functools

import jax
import jax.numpy as jnp
from jax import lax
from jax.experimental import pallas as pl
from jax.experimental.pallas import tpu as pltpu
from jax.experimental.pallas import tpu_sc as plsc

_B = 16384
_E = 128
_NC = 2
_NS = 16
_NW = _NC * _NS
_PW = _B // _NW
_C = 128
_NCHUNK = _PW // _C
_NBUF = 3


def _sc_body(xt_hbm, xc_hbm, wt_hbm, wc_hbm, out_hbm,
             idxt_v, idxc_v, rt0, rc0, rt1, rc1, rt2, rc2, out_v,
             sem0, sem1, sem2):
    wid = lax.axis_index("s") * _NC + lax.axis_index("c")
    base = wid * _PW

    rows = [(rt0, rc0), (rt1, rc1), (rt2, rc2)]
    sems = [sem0, sem1, sem2]

    def start(c):
        rt, rc = rows[c % _NBUF]
        sem = sems[c % _NBUF]
        cp_t = pltpu.async_copy(wt_hbm.at[idxt_v.at[pl.ds(c * _C, _C)]], rt, sem)
        cp_c = pltpu.async_copy(wc_hbm.at[idxc_v.at[pl.ds(c * _C, _C)]], rc, sem)
        return cp_t, cp_c

    pltpu.sync_copy(xt_hbm.at[pl.ds(base, _C)], idxt_v.at[pl.ds(0, _C)])
    pltpu.sync_copy(xc_hbm.at[pl.ds(base, _C)], idxc_v.at[pl.ds(0, _C)])
    inflight = [start(0)]
    pltpu.sync_copy(xt_hbm.at[pl.ds(base + _C, _PW - _C)],
                    idxt_v.at[pl.ds(_C, _PW - _C)])
    pltpu.sync_copy(xc_hbm.at[pl.ds(base + _C, _PW - _C)],
                    idxc_v.at[pl.ds(_C, _PW - _C)])
    inflight.append(start(1))

    lane = lax.iota(jnp.int32, 16)
    perms = [lane ^ jnp.int32(1 << p) for p in (3, 2, 1, 0)]

    for c in range(_NCHUNK):
        for cp in inflight[0]:
            cp.wait()
        inflight.pop(0)
        if c + 2 < _NCHUNK:
            inflight.append(start(c + 2))
        rt, rc = rows[c % _NBUF]

        def grp_body(g, carry, rt=rt, rc=rc, c=c):
            def tok_body(i, res, rt=rt, rc=rc):
                t = g * 16 + i
                s = rt[t, pl.ds(0, 16)] * rc[t, pl.ds(0, 16)]
                for k in range(1, _E // 16):
                    s = s + (rt[t, pl.ds(k * 16, 16)]
                             * rc[t, pl.ds(k * 16, 16)])
                for pv in perms:
                    s = s + s.at[pv].get(mode="promise_in_bounds")
                return jnp.where(lane == i, s, res)

            res = lax.fori_loop(0, 16, tok_body,
                                jnp.zeros((16,), jnp.float32), unroll=4)
            out_v[pl.ds(c * _C + g * 16, 16)] = res
            return carry

        lax.fori_loop(0, _C // 16, grp_body, 0)

    pltpu.sync_copy(out_v, out_hbm.at[pl.ds(base, _PW)])


@jax.jit
def _skipgram_sc(xt, xc, wt, wc):
    f = functools.partial(
        pl.kernel,
        out_type=jax.ShapeDtypeStruct((_B,), jnp.float32),
        mesh=plsc.VectorSubcoreMesh(core_axis_name="c", subcore_axis_name="s"),
        scratch_types=[
            pltpu.VMEM((_PW,), jnp.int32),
            pltpu.VMEM((_PW,), jnp.int32),
            pltpu.VMEM((_C, _E), jnp.float32),
            pltpu.VMEM((_C, _E), jnp.float32),
            pltpu.VMEM((_C, _E), jnp.float32),
            pltpu.VMEM((_C, _E), jnp.float32),
            pltpu.VMEM((_C, _E), jnp.float32),
            pltpu.VMEM((_C, _E), jnp.float32),
            pltpu.VMEM((_PW,), jnp.float32),
            pltpu.SemaphoreType.DMA,
            pltpu.SemaphoreType.DMA,
            pltpu.SemaphoreType.DMA,
        ],
    )(_sc_body)
    return f(xt, xc, wt, wc)


def kernel(X_target, X_context, W_target, W_context):
    xt = X_target.astype(jnp.int32)
    xc = X_context.astype(jnp.int32)
    return _skipgram_sc(xt, xc, W_target, W_context)


# speedup vs baseline: 1.0707x; 1.0138x over previous
import functools

import jax
import jax.numpy as jnp
from jax import lax
from jax.experimental import pallas as pl
from jax.experimental.pallas import tpu as pltpu
from jax.experimental.pallas import tpu_sc as plsc

_B = 16384
_E = 128
_NC = 2
_NS = 16
_NW = _NC * _NS
_PW = _B // _NW
_C = 128
_NCHUNK = _PW // _C
_NBUF = 3


def _sc_body(xt_hbm, xc_hbm, wt_hbm, wc_hbm, out_hbm,
             idxt_v, idxc_v, rt0, rc0, rt1, rc1, rt2, rc2, out_v,
             sem0, sem1, sem2):
    wid = lax.axis_index("s") * _NC + lax.axis_index("c")
    base = wid * _PW

    rows = [(rt0, rc0), (rt1, rc1), (rt2, rc2)]
    sems = [sem0, sem1, sem2]

    def start(c):
        rt, rc = rows[c % _NBUF]
        sem = sems[c % _NBUF]
        cp_t = pltpu.async_copy(wt_hbm.at[idxt_v.at[pl.ds(c * _C, _C)]], rt, sem)
        cp_c = pltpu.async_copy(wc_hbm.at[idxc_v.at[pl.ds(c * _C, _C)]], rc, sem)
        return cp_t, cp_c

    cp0t = pltpu.async_copy(xt_hbm.at[pl.ds(base, _C)],
                            idxt_v.at[pl.ds(0, _C)], sem0)
    cp0c = pltpu.async_copy(xc_hbm.at[pl.ds(base, _C)],
                            idxc_v.at[pl.ds(0, _C)], sem0)
    cp0t.wait()
    cp0c.wait()
    inflight = [start(0)]
    cp1t = pltpu.async_copy(xt_hbm.at[pl.ds(base + _C, _PW - _C)],
                            idxt_v.at[pl.ds(_C, _PW - _C)], sem1)
    cp1c = pltpu.async_copy(xc_hbm.at[pl.ds(base + _C, _PW - _C)],
                            idxc_v.at[pl.ds(_C, _PW - _C)], sem1)
    cp1t.wait()
    cp1c.wait()
    inflight.append(start(1))

    lane = lax.iota(jnp.int32, 16)
    perms = [lane ^ jnp.int32(1 << p) for p in (3, 2, 1, 0)]

    for c in range(_NCHUNK):
        for cp in inflight[0]:
            cp.wait()
        inflight.pop(0)
        if c + 2 < _NCHUNK:
            inflight.append(start(c + 2))
        rt, rc = rows[c % _NBUF]

        def grp_body(g, carry, rt=rt, rc=rc, c=c):
            def tok_body(i, res, rt=rt, rc=rc):
                t = g * 16 + i
                s = rt[t, pl.ds(0, 16)] * rc[t, pl.ds(0, 16)]
                for k in range(1, _E // 16):
                    s = s + (rt[t, pl.ds(k * 16, 16)]
                             * rc[t, pl.ds(k * 16, 16)])
                for pv in perms:
                    s = s + s.at[pv].get(mode="promise_in_bounds")
                return jnp.where(lane == i, s, res)

            res = lax.fori_loop(0, 16, tok_body,
                                jnp.zeros((16,), jnp.float32), unroll=4)
            out_v[pl.ds(c * _C + g * 16, 16)] = res
            return carry

        lax.fori_loop(0, _C // 16, grp_body, 0)

    pltpu.sync_copy(out_v, out_hbm.at[pl.ds(base, _PW)])


@jax.jit
def _skipgram_sc(xt, xc, wt, wc):
    f = functools.partial(
        pl.kernel,
        out_type=jax.ShapeDtypeStruct((_B,), jnp.float32),
        mesh=plsc.VectorSubcoreMesh(core_axis_name="c", subcore_axis_name="s"),
        scratch_types=[
            pltpu.VMEM((_PW,), jnp.int32),
            pltpu.VMEM((_PW,), jnp.int32),
            pltpu.VMEM((_C, _E), jnp.float32),
            pltpu.VMEM((_C, _E), jnp.float32),
            pltpu.VMEM((_C, _E), jnp.float32),
            pltpu.VMEM((_C, _E), jnp.float32),
            pltpu.VMEM((_C, _E), jnp.float32),
            pltpu.VMEM((_C, _E), jnp.float32),
            pltpu.VMEM((_PW,), jnp.float32),
            pltpu.SemaphoreType.DMA,
            pltpu.SemaphoreType.DMA,
            pltpu.SemaphoreType.DMA,
        ],
    )(_sc_body)
    return f(xt, xc, wt, wc)


def kernel(X_target, X_context, W_target, W_context):
    xt = X_target.astype(jnp.int32)
    xc = X_context.astype(jnp.int32)
    return _skipgram_sc(xt, xc, W_target, W_context)
